# baseline (speedup 1.0000x reference)
import functools

import jax
import jax.numpy as jnp
from jax import lax
from jax.experimental import pallas as pl
from jax.experimental.pallas import tpu as pltpu
from jax.experimental.pallas import tpu_sc as plsc

N = 50000
E = 800000
HID = 64
HALF = 32
NPAD = 50048
ROWS_PER_TILE = N // 16
CHUNK = 125
NROW = E // CHUNK
SUPER = 16
ROWS_PER_TILE_E = NROW // 16
NSUPER = ROWS_PER_TILE_E // SUPER
BN = 400
GRID = N // BN
EPS_BN = 1e-5

_mesh = plsc.VectorSubcoreMesh(core_axis_name="c", subcore_axis_name="s")


@functools.partial(
    pl.kernel,
    mesh=_mesh,
    out_type=(
        jax.ShapeDtypeStruct((NPAD,), jnp.float32),
        jax.ShapeDtypeStruct((NPAD,), jnp.float32),
    ),
    scratch_types=[
        pltpu.VMEM_SHARED((NPAD,), jnp.float32),
        pltpu.VMEM((2, SUPER, CHUNK), jnp.int32),
        pltpu.VMEM((CHUNK,), jnp.float32),
        pltpu.VMEM((3128,), jnp.float32),
        pltpu.SemaphoreType.DMA((4,)),
        pltpu.SemaphoreType.DMA((2,)),
    ],
)
def _deg_kernel(ones_hbm, sdst_hbm, tdst_hbm, degs_out, degt_out,
                acc, idx_st, ones_v, stage_v, s_sem, st_sem):
    c = lax.axis_index("c")
    s = lax.axis_index("s")
    pltpu.sync_copy(ones_hbm.at[pl.ds(0, CHUNK)], ones_v)
    pltpu.sync_copy(ones_hbm.at[pl.ds(s * 3128, 3128)], stage_v)
    pltpu.sync_copy(stage_v, acc.at[pl.ds(s * 3128, 3128)])
    plsc.subcore_barrier()

    def count(dst_hbm):
        pltpu.sync_copy(dst_hbm.at[pl.ds(s * ROWS_PER_TILE_E, SUPER)],
                        idx_st.at[0])

        def body(k, carry):
            p = k % 2
            q = (k + 1) % 2
            r0n = jnp.minimum(s * ROWS_PER_TILE_E + (k + 1) * SUPER,
                              NROW - SUPER)
            h1 = pltpu.async_copy(dst_hbm.at[pl.ds(r0n, SUPER)],
                                  idx_st.at[q], st_sem.at[0])
            sc = {}
            for j in range(SUPER):
                if j >= 4:
                    sc[j - 4].wait()
                sc[j] = pltpu.async_copy(ones_v, acc.at[idx_st.at[p, j]],
                                         s_sem.at[j % 4], add=True)
            for t in range(SUPER - 4, SUPER):
                sc[t].wait()
            h1.wait()
            return carry
        lax.fori_loop(0, NSUPER, body, 0)

    @pl.when(c == 0)
    def _():
        count(sdst_hbm)

    @pl.when(c == 1)
    def _():
        count(tdst_hbm)

    plsc.subcore_barrier()

    pltpu.sync_copy(acc.at[pl.ds(s * 3128, 3128)], stage_v)

    @pl.when(c == 0)
    def _():
        pltpu.sync_copy(stage_v, degs_out.at[pl.ds(s * 3128, 3128)])

    @pl.when(c == 1)
    def _():
        pltpu.sync_copy(stage_v, degt_out.at[pl.ds(s * 3128, 3128)])


@functools.partial(
    pl.kernel,
    mesh=_mesh,
    compiler_params=pltpu.CompilerParams(use_tc_tiling_on_sc=False),
    out_type=jax.ShapeDtypeStruct((2 * N, HALF), jnp.float32),
    scratch_types=[
        pltpu.VMEM_SHARED((N, HALF), jnp.float32),
        pltpu.VMEM((2, SUPER, CHUNK), jnp.int32),
        pltpu.VMEM((2, SUPER, CHUNK), jnp.int32),
        pltpu.VMEM((4, CHUNK, HALF), jnp.float32),
        pltpu.SemaphoreType.DMA((4,)),
        pltpu.SemaphoreType.DMA((4,)),
        pltpu.SemaphoreType.DMA((2,)),
    ],
)
def _prop_kernel(u_hbm, src_hbm, dst_hbm, out_hbm,
                 acc, src_st, dst_st, rows, g_sem, s_sem, st_sem):
    c = lax.axis_index("c")
    s = lax.axis_index("s")
    base = s * 3128

    @pl.when(s < 15)
    def _():
        pltpu.sync_copy(u_hbm.at[pl.ds(c * N + base, 3128)],
                        acc.at[pl.ds(base, 3128)])

    @pl.when(s == 15)
    def _():
        pltpu.sync_copy(u_hbm.at[pl.ds(c * N + base, 3080)],
                        acc.at[pl.ds(base, 3080)])

    plsc.subcore_barrier()

    r0 = s * ROWS_PER_TILE_E
    pltpu.sync_copy(src_hbm.at[c, pl.ds(r0, SUPER)], src_st.at[0])
    pltpu.sync_copy(dst_hbm.at[pl.ds(r0, SUPER)], dst_st.at[0])

    def body(k, carry):
        p = k % 2
        q = (k + 1) % 2
        r0n = jnp.minimum(s * ROWS_PER_TILE_E + (k + 1) * SUPER,
                          NROW - SUPER)
        h1 = pltpu.async_copy(src_hbm.at[c, pl.ds(r0n, SUPER)],
                              src_st.at[q], st_sem.at[0])
        h2 = pltpu.async_copy(dst_hbm.at[pl.ds(r0n, SUPER)],
                              dst_st.at[q], st_sem.at[1])

        def gath(j):
            return pltpu.async_copy(u_hbm.at[src_st.at[p, j]],
                                    rows.at[j % 4], g_sem.at[j % 4])

        g = {j: gath(j) for j in range(3)}
        sc = {}
        for j in range(SUPER):
            if j >= 1:
                sc[j - 1].wait()
            if j + 3 < SUPER:
                g[j + 3] = gath(j + 3)
            g[j].wait()
            sc[j] = pltpu.async_copy(rows.at[j % 4],
                                     acc.at[dst_st.at[p, j]],
                                     s_sem.at[j % 4], add=True)
        sc[SUPER - 1].wait()
        h1.wait()
        h2.wait()
        return carry

    lax.fori_loop(0, NSUPER, body, 0)
    plsc.subcore_barrier()

    @pl.when(s < 15)
    def _():
        pltpu.sync_copy(acc.at[pl.ds(base, 3128)],
                        out_hbm.at[pl.ds(c * N + base, 3128)])

    @pl.when(s == 15)
    def _():
        pltpu.sync_copy(acc.at[pl.ds(base, 3080)],
                        out_hbm.at[pl.ds(c * N + base, 3080)])


SUPER1 = 20
ROWS_PER_TILE_E1 = NROW // 32
NSUPER1 = ROWS_PER_TILE_E1 // SUPER1


@functools.partial(
    pl.kernel,
    mesh=_mesh,
    compiler_params=pltpu.CompilerParams(use_tc_tiling_on_sc=False),
    out_type=jax.ShapeDtypeStruct((2, N, 16), jnp.float32),
    scratch_types=[
        pltpu.VMEM_SHARED((N, 16), jnp.float32),
        pltpu.VMEM((2, SUPER1, CHUNK), jnp.int32),
        pltpu.VMEM((2, SUPER1, CHUNK), jnp.int32),
        pltpu.VMEM((4, CHUNK, 16), jnp.float32),
        pltpu.SemaphoreType.DMA((4,)),
        pltpu.SemaphoreType.DMA((4,)),
        pltpu.SemaphoreType.DMA((2,)),
    ],
)
def _prop16_kernel(u_hbm, zeros_hbm, src_hbm, dst_hbm, out_hbm,
                   acc, src_st, dst_st, rows, g_sem, s_sem, st_sem):
    c = lax.axis_index("c")
    s = lax.axis_index("s")
    base = s * 3128

    def init(tbl):
        @pl.when(s < 15)
        def _():
            pltpu.sync_copy(tbl.at[pl.ds(base, 3128)],
                            acc.at[pl.ds(base, 3128)])

        @pl.when(s == 15)
        def _():
            pltpu.sync_copy(tbl.at[pl.ds(base, 3080)],
                            acc.at[pl.ds(base, 3080)])

    @pl.when(c == 0)
    def _():
        init(u_hbm)

    @pl.when(c == 1)
    def _():
        init(zeros_hbm)

    plsc.subcore_barrier()

    r0 = c * (NROW // 2) + s * ROWS_PER_TILE_E1
    pltpu.sync_copy(src_hbm.at[pl.ds(r0, SUPER1)], src_st.at[0])
    pltpu.sync_copy(dst_hbm.at[pl.ds(r0, SUPER1)], dst_st.at[0])

    def body(k, carry):
        p = k % 2
        q = (k + 1) % 2
        r0n = jnp.minimum(r0 + (k + 1) * SUPER1, NROW - SUPER1)
        h1 = pltpu.async_copy(src_hbm.at[pl.ds(r0n, SUPER1)],
                              src_st.at[q], st_sem.at[0])
        h2 = pltpu.async_copy(dst_hbm.at[pl.ds(r0n, SUPER1)],
                              dst_st.at[q], st_sem.at[1])

        def gath(j):
            return pltpu.async_copy(u_hbm.at[src_st.at[p, j]],
                                    rows.at[j % 4], g_sem.at[j % 4])

        g = {j: gath(j) for j in range(3)}
        sc = {}
        for j in range(SUPER1):
            if j >= 1:
                sc[j - 1].wait()
            if j + 3 < SUPER1:
                g[j + 3] = gath(j + 3)
            g[j].wait()
            sc[j] = pltpu.async_copy(rows.at[j % 4],
                                     acc.at[dst_st.at[p, j]],
                                     s_sem.at[j % 4], add=True)
        sc[SUPER1 - 1].wait()
        h1.wait()
        h2.wait()
        return carry

    lax.fori_loop(0, NSUPER1, body, 0)
    plsc.subcore_barrier()

    @pl.when(s < 15)
    def _():
        pltpu.sync_copy(acc.at[pl.ds(base, 3128)],
                        out_hbm.at[c, pl.ds(base, 3128)])

    @pl.when(s == 15)
    def _():
        pltpu.sync_copy(acc.at[pl.ds(base, 3080)],
                        out_hbm.at[c, pl.ds(base, 3080)])


def _pre_body(x_ref, degs_ref, degt_ref, dinvs_ref, dinvt_ref, u_ref):
    ds = lax.rsqrt(jnp.maximum(degs_ref[...], 1e-12))
    dt = lax.rsqrt(jnp.maximum(degt_ref[...], 1e-12))
    dinvs_ref[...] = ds
    dinvt_ref[...] = dt
    u_ref[...] = x_ref[...] * ds


def _accum_stats(z, sums_ref):
    j = pl.program_id(1)
    zr = z.reshape(BN // 8, 8, HID)
    s1 = jnp.sum(zr, axis=0)
    s2 = jnp.sum(zr * zr, axis=0)

    @pl.when(j == 0)
    def _():
        sums_ref[0:8] = s1
        sums_ref[8:16] = s2

    @pl.when(j > 0)
    def _():
        sums_ref[0:8] += s1
        sums_ref[8:16] += s2


def _bn_relu(z, sums_ref, g_ref, be_ref):
    m = jnp.sum(sums_ref[0:8, :], axis=0, keepdims=True) / N
    ex2 = jnp.sum(sums_ref[8:16, :], axis=0, keepdims=True) / N
    var = ex2 - m * m
    y = (z - m) * lax.rsqrt(var + EPS_BN) * g_ref[...] + be_ref[...]
    return jnp.maximum(y, 0.0)


def _mid1_body(acc_ref, dinv_in_ref, dinv_out_ref,
               g_ref, be_ref, w1_ref, w_ref, u_ref, sums_ref):
    p = pl.program_id(0)
    t = (acc_ref[0] + acc_ref[1]) * dinv_in_ref[...]
    z = jnp.dot(t, w1_ref[...], preferred_element_type=jnp.float32,
                precision=lax.Precision.HIGHEST)

    @pl.when(p == 0)
    def _():
        _accum_stats(z, sums_ref)

    @pl.when(p == 1)
    def _():
        y = _bn_relu(z, sums_ref, g_ref, be_ref)
        h = jnp.dot(y, w_ref[...], preferred_element_type=jnp.float32,
                    precision=lax.Precision.HIGHEST)
        u = h * dinv_out_ref[...]
        u_ref[0] = u[:, :HALF]
        u_ref[1] = u[:, HALF:]


def _mid_body(lo_ref, hi_ref, dinv_in_ref, dinv_out_ref,
              g_ref, be_ref, w_ref, u_ref, sums_ref):
    p = pl.program_id(0)
    z = (jnp.concatenate([lo_ref[...], hi_ref[...]], axis=1)
         * dinv_in_ref[...])

    @pl.when(p == 0)
    def _():
        _accum_stats(z, sums_ref)

    @pl.when(p == 1)
    def _():
        y = _bn_relu(z, sums_ref, g_ref, be_ref)
        h = jnp.dot(y, w_ref[...], preferred_element_type=jnp.float32,
                    precision=lax.Precision.HIGHEST)
        u = h * dinv_out_ref[...]
        u_ref[0] = u[:, :HALF]
        u_ref[1] = u[:, HALF:]


def _final_body(lo_ref, hi_ref, dinv_in_ref,
                g_ref, be_ref, wf_ref, bf_ref, out_ref, sums_ref):
    p = pl.program_id(0)
    z = (jnp.concatenate([lo_ref[...], hi_ref[...]], axis=1)
         * dinv_in_ref[...])

    @pl.when(p == 0)
    def _():
        _accum_stats(z, sums_ref)

    @pl.when(p == 1)
    def _():
        y = _bn_relu(z, sums_ref, g_ref, be_ref)
        out_ref[...] = jnp.dot(y, wf_ref[...],
                               preferred_element_type=jnp.float32,
                               precision=lax.Precision.HIGHEST) + bf_ref[...]


def _rows(shape):
    return pl.BlockSpec(shape, lambda *i: (i[-1],) + (0,) * (len(shape) - 1))


def _whole(shape):
    return pl.BlockSpec(shape, lambda *i: (0,) * len(shape))


_pre_call = pl.pallas_call(
    _pre_body,
    grid=(GRID,),
    in_specs=[_rows((BN, 16)), _rows((BN, 1)), _rows((BN, 1))],
    out_specs=(_rows((BN, 1)), _rows((BN, 1)), _rows((BN, 16))),
    out_shape=(jax.ShapeDtypeStruct((N, 1), jnp.float32),
               jax.ShapeDtypeStruct((N, 1), jnp.float32),
               jax.ShapeDtypeStruct((N, 16), jnp.float32)),
)

_mid1_call = pl.pallas_call(
    _mid1_body,
    grid=(2, GRID),
    in_specs=[pl.BlockSpec((2, BN, 16), lambda p, j: (0, j, 0)),
              _rows((BN, 1)), _rows((BN, 1)),
              _whole((1, HID)), _whole((1, HID)), _whole((16, HID)),
              _whole((HID, HID))],
    out_specs=pl.BlockSpec((2, BN, HALF), lambda p, j: (0, j, 0)),
    out_shape=jax.ShapeDtypeStruct((2, N, HALF), jnp.float32),
    scratch_shapes=[pltpu.VMEM((16, HID), jnp.float32)],
)

_mid_call = pl.pallas_call(
    _mid_body,
    grid=(2, GRID),
    in_specs=[_rows((BN, HALF)), _rows((BN, HALF)), _rows((BN, 1)),
              _rows((BN, 1)), _whole((1, HID)), _whole((1, HID)),
              _whole((HID, HID))],
    out_specs=pl.BlockSpec((2, BN, HALF), lambda p, j: (0, j, 0)),
    out_shape=jax.ShapeDtypeStruct((2, N, HALF), jnp.float32),
    scratch_shapes=[pltpu.VMEM((16, HID), jnp.float32)],
)

_final_call = pl.pallas_call(
    _final_body,
    grid=(2, GRID),
    in_specs=[_rows((BN, HALF)), _rows((BN, HALF)), _rows((BN, 1)),
              _whole((1, HID)), _whole((1, HID)),
              _whole((HID, 4)), _whole((1, 4))],
    out_specs=_rows((BN, 4)),
    out_shape=jax.ShapeDtypeStruct((N, 4), jnp.float32),
    scratch_shapes=[pltpu.VMEM((16, HID), jnp.float32)],
)


def kernel(x, spatial_edge_index, temporal_edge_index,
           W1, b1, g1, be1, W2, b2, g2, be2,
           W3, b3, g3, be3, W4, b4, g4, be4, Wf, bf):
    f32 = jnp.float32
    ones = jnp.ones((NPAD,), f32)
    sdst2 = spatial_edge_index[1].reshape(NROW, CHUNK)
    tdst2 = temporal_edge_index[1].reshape(NROW, CHUNK)
    ssrc3 = jnp.stack([spatial_edge_index[0],
                       spatial_edge_index[0] + N]).reshape(2, NROW, CHUNK)
    tsrc3 = jnp.stack([temporal_edge_index[0],
                       temporal_edge_index[0] + N]).reshape(2, NROW, CHUNK)

    deg_s, deg_t = _deg_kernel(ones, sdst2, tdst2)
    x_p = jnp.pad(x, ((0, 0), (0, 16 - x.shape[1])))
    W1p = jnp.pad(W1, ((0, 16 - W1.shape[0]), (0, 0)))
    dinv_s, dinv_t, u1p = _pre_call(x_p,
                                    deg_s[:N].reshape(N, 1),
                                    deg_t[:N].reshape(N, 1))

    ssrc2 = spatial_edge_index[0].reshape(NROW, CHUNK)
    zeros16 = jnp.zeros((N, 16), f32)
    acc3 = _prop16_kernel(u1p, zeros16, ssrc2, sdst2)
    u = _mid1_call(acc3, dinv_s, dinv_s,
                   g1.reshape(1, HID), be1.reshape(1, HID), W1p, W2)

    layers = [
        (ssrc3, sdst2, dinv_s, dinv_t, g2, be2, W3),
        (tsrc3, tdst2, dinv_t, dinv_t, g3, be3, W4),
    ]
    for src3, dst2, dv_in, dv_out, g, be, w_next in layers:
        acc = _prop_kernel(u.reshape(2 * N, HALF), src3, dst2)
        lo, hi = acc[:N], acc[N:]
        u = _mid_call(lo, hi, dv_in, dv_out,
                      g.reshape(1, HID), be.reshape(1, HID), w_next)

    acc = _prop_kernel(u.reshape(2 * N, HALF), tsrc3, tdst2)
    lo, hi = acc[:N], acc[N:]
    return _final_call(lo, hi, dinv_t,
                       g4.reshape(1, HID), be4.reshape(1, HID),
                       Wf, bf.reshape(1, 4))


# speedup vs baseline: 1.0396x; 1.0396x over previous
import functools

import jax
import jax.numpy as jnp
from jax import lax
from jax.experimental import pallas as pl
from jax.experimental.pallas import tpu as pltpu
from jax.experimental.pallas import tpu_sc as plsc

N = 50000
E = 800000
HID = 64
HALF = 32
NPAD = 50048
ROWS_PER_TILE = N // 16
CHUNK = 125
NROW = E // CHUNK
SUPER = 16
ROWS_PER_TILE_E = NROW // 16
NSUPER = ROWS_PER_TILE_E // SUPER
BN = 400
GRID = N // BN
EPS_BN = 1e-5

_mesh = plsc.VectorSubcoreMesh(core_axis_name="c", subcore_axis_name="s")


@functools.partial(
    pl.kernel,
    mesh=_mesh,
    out_type=(
        jax.ShapeDtypeStruct((NPAD,), jnp.float32),
        jax.ShapeDtypeStruct((NPAD,), jnp.float32),
    ),
    scratch_types=[
        pltpu.VMEM_SHARED((NPAD,), jnp.float32),
        pltpu.VMEM((2, SUPER, CHUNK), jnp.int32),
        pltpu.VMEM((CHUNK,), jnp.float32),
        pltpu.VMEM((3128,), jnp.float32),
        pltpu.SemaphoreType.DMA((4,)),
        pltpu.SemaphoreType.DMA((2,)),
    ],
)
def _deg_kernel(ones_hbm, sdst_hbm, tdst_hbm, degs_out, degt_out,
                acc, idx_st, ones_v, stage_v, s_sem, st_sem):
    c = lax.axis_index("c")
    s = lax.axis_index("s")
    pltpu.sync_copy(ones_hbm.at[pl.ds(0, CHUNK)], ones_v)
    pltpu.sync_copy(ones_hbm.at[pl.ds(s * 3128, 3128)], stage_v)
    pltpu.sync_copy(stage_v, acc.at[pl.ds(s * 3128, 3128)])
    plsc.subcore_barrier()

    def count(dst_hbm):
        pltpu.sync_copy(dst_hbm.at[pl.ds(s * ROWS_PER_TILE_E, SUPER)],
                        idx_st.at[0])

        def body(k, carry):
            p = k % 2
            q = (k + 1) % 2
            r0n = jnp.minimum(s * ROWS_PER_TILE_E + (k + 1) * SUPER,
                              NROW - SUPER)
            h1 = pltpu.async_copy(dst_hbm.at[pl.ds(r0n, SUPER)],
                                  idx_st.at[q], st_sem.at[0])
            sc = {}
            for j in range(SUPER):
                if j >= 4:
                    sc[j - 4].wait()
                sc[j] = pltpu.async_copy(ones_v, acc.at[idx_st.at[p, j]],
                                         s_sem.at[j % 4], add=True)
            for t in range(SUPER - 4, SUPER):
                sc[t].wait()
            h1.wait()
            return carry
        lax.fori_loop(0, NSUPER, body, 0)

    @pl.when(c == 0)
    def _():
        count(sdst_hbm)

    @pl.when(c == 1)
    def _():
        count(tdst_hbm)

    plsc.subcore_barrier()

    pltpu.sync_copy(acc.at[pl.ds(s * 3128, 3128)], stage_v)

    @pl.when(c == 0)
    def _():
        pltpu.sync_copy(stage_v, degs_out.at[pl.ds(s * 3128, 3128)])

    @pl.when(c == 1)
    def _():
        pltpu.sync_copy(stage_v, degt_out.at[pl.ds(s * 3128, 3128)])


@functools.partial(
    pl.kernel,
    mesh=_mesh,
    compiler_params=pltpu.CompilerParams(use_tc_tiling_on_sc=False),
    out_type=jax.ShapeDtypeStruct((2 * N, HALF), jnp.float32),
    scratch_types=[
        pltpu.VMEM_SHARED((N, HALF), jnp.float32),
        pltpu.VMEM((2, SUPER, CHUNK), jnp.int32),
        pltpu.VMEM((2, SUPER, CHUNK), jnp.int32),
        pltpu.VMEM((4, CHUNK, HALF), jnp.float32),
        pltpu.SemaphoreType.DMA((4,)),
        pltpu.SemaphoreType.DMA((4,)),
        pltpu.SemaphoreType.DMA((2,)),
    ],
)
def _prop_kernel(u_hbm, src_hbm, dst_hbm, out_hbm,
                 acc, src_st, dst_st, rows, g_sem, s_sem, st_sem):
    c = lax.axis_index("c")
    s = lax.axis_index("s")
    base = s * 3128

    @pl.when(s < 15)
    def _():
        pltpu.sync_copy(u_hbm.at[pl.ds(c * N + base, 3128)],
                        acc.at[pl.ds(base, 3128)])

    @pl.when(s == 15)
    def _():
        pltpu.sync_copy(u_hbm.at[pl.ds(c * N + base, 3080)],
                        acc.at[pl.ds(base, 3080)])

    plsc.subcore_barrier()

    r0 = s * ROWS_PER_TILE_E
    pltpu.sync_copy(src_hbm.at[c, pl.ds(r0, SUPER)], src_st.at[0])
    pltpu.sync_copy(dst_hbm.at[pl.ds(r0, SUPER)], dst_st.at[0])

    def body(k, carry):
        p = k % 2
        q = (k + 1) % 2
        r0n = jnp.minimum(s * ROWS_PER_TILE_E + (k + 1) * SUPER,
                          NROW - SUPER)
        h1 = pltpu.async_copy(src_hbm.at[c, pl.ds(r0n, SUPER)],
                              src_st.at[q], st_sem.at[0])
        h2 = pltpu.async_copy(dst_hbm.at[pl.ds(r0n, SUPER)],
                              dst_st.at[q], st_sem.at[1])

        def gath(j):
            return pltpu.async_copy(u_hbm.at[src_st.at[p, j]],
                                    rows.at[j % 4], g_sem.at[j % 4])

        g = {j: gath(j) for j in range(3)}
        sc = {}
        for j in range(SUPER):
            if j >= 1:
                sc[j - 1].wait()
            if j + 3 < SUPER:
                g[j + 3] = gath(j + 3)
            g[j].wait()
            sc[j] = pltpu.async_copy(rows.at[j % 4],
                                     acc.at[dst_st.at[p, j]],
                                     s_sem.at[j % 4], add=True)
        sc[SUPER - 1].wait()
        h1.wait()
        h2.wait()
        return carry

    lax.fori_loop(0, NSUPER, body, 0)
    plsc.subcore_barrier()

    @pl.when(s < 15)
    def _():
        pltpu.sync_copy(acc.at[pl.ds(base, 3128)],
                        out_hbm.at[pl.ds(c * N + base, 3128)])

    @pl.when(s == 15)
    def _():
        pltpu.sync_copy(acc.at[pl.ds(base, 3080)],
                        out_hbm.at[pl.ds(c * N + base, 3080)])


SUPER1 = 20
ROWS_PER_TILE_E1 = NROW // 32
NSUPER1 = ROWS_PER_TILE_E1 // SUPER1


@functools.partial(
    pl.kernel,
    mesh=_mesh,
    compiler_params=pltpu.CompilerParams(use_tc_tiling_on_sc=False),
    out_type=jax.ShapeDtypeStruct((2, N, 16), jnp.float32),
    scratch_types=[
        pltpu.VMEM_SHARED((N, 16), jnp.float32),
        pltpu.VMEM((2, SUPER1, CHUNK), jnp.int32),
        pltpu.VMEM((2, SUPER1, CHUNK), jnp.int32),
        pltpu.VMEM((4, CHUNK, 16), jnp.float32),
        pltpu.SemaphoreType.DMA((4,)),
        pltpu.SemaphoreType.DMA((4,)),
        pltpu.SemaphoreType.DMA((2,)),
    ],
)
def _prop16_kernel(u_hbm, zeros_hbm, src_hbm, dst_hbm, out_hbm,
                   acc, src_st, dst_st, rows, g_sem, s_sem, st_sem):
    c = lax.axis_index("c")
    s = lax.axis_index("s")
    base = s * 3128

    def init(tbl):
        @pl.when(s < 15)
        def _():
            pltpu.sync_copy(tbl.at[pl.ds(base, 3128)],
                            acc.at[pl.ds(base, 3128)])

        @pl.when(s == 15)
        def _():
            pltpu.sync_copy(tbl.at[pl.ds(base, 3080)],
                            acc.at[pl.ds(base, 3080)])

    @pl.when(c == 0)
    def _():
        init(u_hbm)

    @pl.when(c == 1)
    def _():
        init(zeros_hbm)

    plsc.subcore_barrier()

    r0 = c * (NROW // 2) + s * ROWS_PER_TILE_E1
    pltpu.sync_copy(src_hbm.at[pl.ds(r0, SUPER1)], src_st.at[0])
    pltpu.sync_copy(dst_hbm.at[pl.ds(r0, SUPER1)], dst_st.at[0])

    def body(k, carry):
        p = k % 2
        q = (k + 1) % 2
        r0n = jnp.minimum(r0 + (k + 1) * SUPER1, NROW - SUPER1)
        h1 = pltpu.async_copy(src_hbm.at[pl.ds(r0n, SUPER1)],
                              src_st.at[q], st_sem.at[0])
        h2 = pltpu.async_copy(dst_hbm.at[pl.ds(r0n, SUPER1)],
                              dst_st.at[q], st_sem.at[1])

        def gath(j):
            return pltpu.async_copy(u_hbm.at[src_st.at[p, j]],
                                    rows.at[j % 4], g_sem.at[j % 4])

        g = {j: gath(j) for j in range(3)}
        sc = {}
        for j in range(SUPER1):
            if j >= 1:
                sc[j - 1].wait()
            if j + 3 < SUPER1:
                g[j + 3] = gath(j + 3)
            g[j].wait()
            sc[j] = pltpu.async_copy(rows.at[j % 4],
                                     acc.at[dst_st.at[p, j]],
                                     s_sem.at[j % 4], add=True)
        sc[SUPER1 - 1].wait()
        h1.wait()
        h2.wait()
        return carry

    lax.fori_loop(0, NSUPER1, body, 0)
    plsc.subcore_barrier()

    @pl.when(s < 15)
    def _():
        pltpu.sync_copy(acc.at[pl.ds(base, 3128)],
                        out_hbm.at[c, pl.ds(base, 3128)])

    @pl.when(s == 15)
    def _():
        pltpu.sync_copy(acc.at[pl.ds(base, 3080)],
                        out_hbm.at[c, pl.ds(base, 3080)])


def _pre_body(x_ref, degs_ref, degt_ref, dinvs_ref, dinvt_ref, u_ref):
    ds = lax.rsqrt(jnp.maximum(degs_ref[...], 1e-12))
    dt = lax.rsqrt(jnp.maximum(degt_ref[...], 1e-12))
    dinvs_ref[...] = ds
    dinvt_ref[...] = dt
    u_ref[...] = x_ref[...] * ds


def _accum_stats(z, out_ref):
    j = pl.program_id(0)
    zr = z.reshape(BN // 8, 8, HID)
    s1 = jnp.sum(zr, axis=0)
    s2 = jnp.sum(zr * zr, axis=0)

    @pl.when(j == 0)
    def _():
        out_ref[0:8] = s1
        out_ref[8:16] = s2

    @pl.when(j > 0)
    def _():
        out_ref[0:8] += s1
        out_ref[8:16] += s2


def _bn_relu(z, sums_ref, g_ref, be_ref):
    m = jnp.sum(sums_ref[0:8, :], axis=0, keepdims=True) / N
    ex2 = jnp.sum(sums_ref[8:16, :], axis=0, keepdims=True) / N
    var = ex2 - m * m
    y = (z - m) * lax.rsqrt(var + EPS_BN) * g_ref[...] + be_ref[...]
    return jnp.maximum(y, 0.0)


def _stats1_body(acc_ref, dinv_ref, w1_ref, out_ref):
    t = (acc_ref[0] + acc_ref[1]) * dinv_ref[...]
    z = jnp.dot(t, w1_ref[...], preferred_element_type=jnp.float32,
                precision=lax.Precision.HIGHEST)
    _accum_stats(z, out_ref)


def _mid1_body(acc_ref, dinv_in_ref, dinv_out_ref, sums_ref,
               g_ref, be_ref, w1_ref, w_ref, u_ref):
    t = (acc_ref[0] + acc_ref[1]) * dinv_in_ref[...]
    z = jnp.dot(t, w1_ref[...], preferred_element_type=jnp.float32,
                precision=lax.Precision.HIGHEST)
    y = _bn_relu(z, sums_ref, g_ref, be_ref)
    h = jnp.dot(y, w_ref[...], preferred_element_type=jnp.float32,
                precision=lax.Precision.HIGHEST)
    u = h * dinv_out_ref[...]
    u_ref[0] = u[:, :HALF]
    u_ref[1] = u[:, HALF:]


def _stats_body(lo_ref, hi_ref, dinv_ref, out_ref):
    z = jnp.concatenate([lo_ref[...], hi_ref[...]], axis=1) * dinv_ref[...]
    _accum_stats(z, out_ref)


def _mid_body(lo_ref, hi_ref, dinv_in_ref, dinv_out_ref, sums_ref,
              g_ref, be_ref, w_ref, u_ref):
    z = (jnp.concatenate([lo_ref[...], hi_ref[...]], axis=1)
         * dinv_in_ref[...])
    y = _bn_relu(z, sums_ref, g_ref, be_ref)
    h = jnp.dot(y, w_ref[...], preferred_element_type=jnp.float32,
                precision=lax.Precision.HIGHEST)
    u = h * dinv_out_ref[...]
    u_ref[0] = u[:, :HALF]
    u_ref[1] = u[:, HALF:]


def _final_body(lo_ref, hi_ref, dinv_in_ref, sums_ref,
                g_ref, be_ref, wf_ref, bf_ref, out_ref):
    z = (jnp.concatenate([lo_ref[...], hi_ref[...]], axis=1)
         * dinv_in_ref[...])
    y = _bn_relu(z, sums_ref, g_ref, be_ref)
    out_ref[...] = jnp.dot(y, wf_ref[...],
                           preferred_element_type=jnp.float32,
                           precision=lax.Precision.HIGHEST) + bf_ref[...]


def _rows(shape):
    return pl.BlockSpec(shape, lambda j: (j,) + (0,) * (len(shape) - 1))


def _whole(shape):
    return pl.BlockSpec(shape, lambda j: (0,) * len(shape))


_pre_call = pl.pallas_call(
    _pre_body,
    grid=(GRID,),
    in_specs=[_rows((BN, 16)), _rows((BN, 1)), _rows((BN, 1))],
    out_specs=(_rows((BN, 1)), _rows((BN, 1)), _rows((BN, 16))),
    out_shape=(jax.ShapeDtypeStruct((N, 1), jnp.float32),
               jax.ShapeDtypeStruct((N, 1), jnp.float32),
               jax.ShapeDtypeStruct((N, 16), jnp.float32)),
)

_stats1_call = pl.pallas_call(
    _stats1_body,
    grid=(GRID,),
    in_specs=[pl.BlockSpec((2, BN, 16), lambda j: (0, j, 0)),
              _rows((BN, 1)), _whole((16, HID))],
    out_specs=_whole((16, HID)),
    out_shape=jax.ShapeDtypeStruct((16, HID), jnp.float32),
)

_mid1_call = pl.pallas_call(
    _mid1_body,
    grid=(GRID,),
    in_specs=[pl.BlockSpec((2, BN, 16), lambda j: (0, j, 0)),
              _rows((BN, 1)), _rows((BN, 1)), _whole((16, HID)),
              _whole((1, HID)), _whole((1, HID)), _whole((16, HID)),
              _whole((HID, HID))],
    out_specs=pl.BlockSpec((2, BN, HALF), lambda j: (0, j, 0)),
    out_shape=jax.ShapeDtypeStruct((2, N, HALF), jnp.float32),
)

_stats_call = pl.pallas_call(
    _stats_body,
    grid=(GRID,),
    in_specs=[_rows((BN, HALF)), _rows((BN, HALF)), _rows((BN, 1))],
    out_specs=_whole((16, HID)),
    out_shape=jax.ShapeDtypeStruct((16, HID), jnp.float32),
)

_mid_call = pl.pallas_call(
    _mid_body,
    grid=(GRID,),
    in_specs=[_rows((BN, HALF)), _rows((BN, HALF)), _rows((BN, 1)),
              _rows((BN, 1)), _whole((16, HID)), _whole((1, HID)),
              _whole((1, HID)), _whole((HID, HID))],
    out_specs=pl.BlockSpec((2, BN, HALF), lambda j: (0, j, 0)),
    out_shape=jax.ShapeDtypeStruct((2, N, HALF), jnp.float32),
)

_final_call = pl.pallas_call(
    _final_body,
    grid=(GRID,),
    in_specs=[_rows((BN, HALF)), _rows((BN, HALF)), _rows((BN, 1)),
              _whole((16, HID)), _whole((1, HID)), _whole((1, HID)),
              _whole((HID, 4)), _whole((1, 4))],
    out_specs=_rows((BN, 4)),
    out_shape=jax.ShapeDtypeStruct((N, 4), jnp.float32),
)


def kernel(x, spatial_edge_index, temporal_edge_index,
           W1, b1, g1, be1, W2, b2, g2, be2,
           W3, b3, g3, be3, W4, b4, g4, be4, Wf, bf):
    f32 = jnp.float32
    ones = jnp.ones((NPAD,), f32)
    sdst2 = spatial_edge_index[1].reshape(NROW, CHUNK)
    tdst2 = temporal_edge_index[1].reshape(NROW, CHUNK)
    ssrc3 = jnp.stack([spatial_edge_index[0],
                       spatial_edge_index[0] + N]).reshape(2, NROW, CHUNK)
    tsrc3 = jnp.stack([temporal_edge_index[0],
                       temporal_edge_index[0] + N]).reshape(2, NROW, CHUNK)

    deg_s, deg_t = _deg_kernel(ones, sdst2, tdst2)
    x_p = jnp.pad(x, ((0, 0), (0, 16 - x.shape[1])))
    W1p = jnp.pad(W1, ((0, 16 - W1.shape[0]), (0, 0)))
    dinv_s, dinv_t, u1p = _pre_call(x_p,
                                    deg_s[:N].reshape(N, 1),
                                    deg_t[:N].reshape(N, 1))

    ssrc2 = spatial_edge_index[0].reshape(NROW, CHUNK)
    zeros16 = jnp.zeros((N, 16), f32)
    acc3 = _prop16_kernel(u1p, zeros16, ssrc2, sdst2)
    sums = _stats1_call(acc3, dinv_s, W1p)
    u = _mid1_call(acc3, dinv_s, dinv_s, sums,
                   g1.reshape(1, HID), be1.reshape(1, HID), W1p, W2)

    layers = [
        (ssrc3, sdst2, dinv_s, dinv_t, g2, be2, W3),
        (tsrc3, tdst2, dinv_t, dinv_t, g3, be3, W4),
    ]
    for src3, dst2, dv_in, dv_out, g, be, w_next in layers:
        acc = _prop_kernel(u.reshape(2 * N, HALF), src3, dst2)
        lo, hi = acc[:N], acc[N:]
        sums = _stats_call(lo, hi, dv_in)
        u = _mid_call(lo, hi, dv_in, dv_out, sums,
                      g.reshape(1, HID), be.reshape(1, HID), w_next)

    acc = _prop_kernel(u.reshape(2 * N, HALF), tsrc3, tdst2)
    lo, hi = acc[:N], acc[N:]
    sums = _stats_call(lo, hi, dinv_t)
    return _final_call(lo, hi, dinv_t, sums,
                       g4.reshape(1, HID), be4.reshape(1, HID),
                       Wf, bf.reshape(1, 4))
